# baseline (device time: 65833 ns/iter reference)
import functools

import jax
import jax.numpy as jnp
from jax import lax
from jax.experimental import pallas as pl
from jax.experimental.pallas import tpu as pltpu

N_DEV = 32
N_STAGES = 5

B, SQ, D = 2, 128, 512
HQ_LOCAL = 4
DH = 64
HD_LOCAL = HQ_LOCAL * DH
ROWS = B * SQ


def kernel(x, Wq, Wk, Wv, Wo):
    def body(x_ref, wq_ref, wk_ref, wv_ref, wo_ref, out_ref,
             acc_ref, recv_ref, send_sems, recv_sems):
        my = lax.axis_index("i")

        barrier = pltpu.get_barrier_semaphore()
        for s in range(N_STAGES):
            partner = jnp.bitwise_xor(my, 1 << s)
            pl.semaphore_signal(
                barrier, inc=1,
                device_id=(partner,), device_id_type=pl.DeviceIdType.MESH,
            )
        pl.semaphore_wait(barrier, N_STAGES)

        xb = x_ref[:].reshape(ROWS, D).astype(jnp.bfloat16)
        q = jnp.dot(xb, wq_ref[:].astype(jnp.bfloat16),
                    preferred_element_type=jnp.float32)
        k = jnp.dot(xb, wk_ref[:].astype(jnp.bfloat16),
                    preferred_element_type=jnp.float32)
        v = jnp.dot(xb, wv_ref[:].astype(jnp.bfloat16),
                    preferred_element_type=jnp.float32)

        rowi = lax.broadcasted_iota(jnp.int32, (ROWS, HD_LOCAL), 0)
        coli = lax.broadcasted_iota(jnp.int32, (ROWS, HD_LOCAL), 1)
        pos = (rowi % SQ).astype(jnp.float32)
        pair = ((coli % DH) // 2).astype(jnp.float32)
        inv = jnp.exp(pair * (-2.0 / DH) * jnp.log(10000.0))
        angle = pos * inv
        cos = jnp.cos(angle)
        sin = jnp.sin(angle)

        ri = lax.broadcasted_iota(jnp.int32, (HD_LOCAL, HD_LOCAL), 0)
        rj = lax.broadcasted_iota(jnp.int32, (HD_LOCAL, HD_LOCAL), 1)
        R = (jnp.where((rj == ri + 1) & (ri % 2 == 0), 1.0, 0.0)
             + jnp.where((rj == ri - 1) & (ri % 2 == 1), -1.0, 0.0))
        q = q * cos + jnp.dot(q, R, preferred_element_type=jnp.float32) * sin
        k = k * cos + jnp.dot(k, R, preferred_element_type=jnp.float32) * sin

        ctx_rows = []
        for b in range(B):
            ctx_heads = []
            for h in range(HQ_LOCAL):
                qs = q[b * SQ:(b + 1) * SQ, h * DH:(h + 1) * DH]
                ks = k[b * SQ:(b + 1) * SQ, h * DH:(h + 1) * DH]
                vs = v[b * SQ:(b + 1) * SQ, h * DH:(h + 1) * DH]
                sc = lax.dot_general(
                    qs, ks, (((1,), (1,)), ((), ())),
                    preferred_element_type=jnp.float32,
                ) * 0.125
                sc = sc - jnp.max(sc, axis=-1, keepdims=True)
                w = jnp.exp(sc)
                w = w / jnp.sum(w, axis=-1, keepdims=True)
                ctx_heads.append(lax.dot_general(
                    w, vs, (((1,), (0,)), ((), ())),
                    preferred_element_type=jnp.float32,
                ))
            ctx_rows.append(jnp.concatenate(ctx_heads, axis=1))
        ctx = jnp.concatenate(ctx_rows, axis=0)

        acc_ref[:, :] = jnp.dot(
            ctx.astype(jnp.bfloat16), wo_ref[:].astype(jnp.bfloat16),
            preferred_element_type=jnp.float32,
        )

        for s in range(N_STAGES):
            partner = jnp.bitwise_xor(my, 1 << s)
            rdma = pltpu.make_async_remote_copy(
                src_ref=acc_ref,
                dst_ref=recv_ref.at[s],
                send_sem=send_sems.at[s],
                recv_sem=recv_sems.at[s],
                device_id=(partner,),
                device_id_type=pl.DeviceIdType.MESH,
            )
            rdma.start()
            rdma.wait()
            acc_ref[:, :] = acc_ref[:, :] + recv_ref[s]

        out_ref[:, :, :] = acc_ref[:, :].reshape(B, SQ, D)

        @functools.partial(
            pl.run_scoped, second_barrier=pltpu.SemaphoreType.REGULAR
        )
        def _(second_barrier):
            for s in range(N_STAGES):
                partner = jnp.bitwise_xor(my, 1 << s)
                pl.semaphore_signal(
                    second_barrier, inc=1,
                    device_id=(partner,), device_id_type=pl.DeviceIdType.MESH,
                )
            pl.semaphore_wait(second_barrier, N_STAGES)

    return pl.pallas_call(
        body,
        out_shape=jax.ShapeDtypeStruct((B, SQ, D), jnp.float32),
        in_specs=[pl.BlockSpec(memory_space=pltpu.VMEM)] * 5,
        out_specs=pl.BlockSpec(memory_space=pltpu.VMEM),
        scratch_shapes=[
            pltpu.VMEM((ROWS, D), jnp.float32),
            pltpu.VMEM((N_STAGES, ROWS, D), jnp.float32),
            pltpu.SemaphoreType.DMA((N_STAGES,)),
            pltpu.SemaphoreType.DMA((N_STAGES,)),
        ],
        compiler_params=pltpu.CompilerParams(collective_id=0),
    )(x, Wq, Wk, Wv, Wo)


# device time: 45443 ns/iter; 1.4487x vs baseline; 1.4487x over previous
import functools

import jax
import jax.numpy as jnp
from jax import lax
from jax.experimental import pallas as pl
from jax.experimental.pallas import tpu as pltpu

N_DEV = 32
N_STAGES = 5

B, SQ, D = 2, 128, 512
HQ_LOCAL = 4
DH = 64
HD_LOCAL = HQ_LOCAL * DH
ROWS = B * SQ


def kernel(x, Wq, Wk, Wv, Wo):
    def body(x_ref, wq_ref, wk_ref, wv_ref, wo_ref, out_ref,
             acc_ref, send_ref, recv_ref, send_sems, recv_sems):
        my = lax.axis_index("i")

        barrier = pltpu.get_barrier_semaphore()
        for s in range(N_STAGES):
            partner = jnp.bitwise_xor(my, 1 << s)
            pl.semaphore_signal(
                barrier, inc=1,
                device_id=(partner,), device_id_type=pl.DeviceIdType.MESH,
            )
        pl.semaphore_wait(barrier, N_STAGES)

        xb = x_ref[:].reshape(ROWS, D).astype(jnp.bfloat16)
        q = jnp.dot(xb, wq_ref[:].astype(jnp.bfloat16),
                    preferred_element_type=jnp.float32)
        k = jnp.dot(xb, wk_ref[:].astype(jnp.bfloat16),
                    preferred_element_type=jnp.float32)
        v = jnp.dot(xb, wv_ref[:].astype(jnp.bfloat16),
                    preferred_element_type=jnp.float32)

        rowi = lax.broadcasted_iota(jnp.int32, (ROWS, HD_LOCAL), 0)
        coli = lax.broadcasted_iota(jnp.int32, (ROWS, HD_LOCAL), 1)
        pos = (rowi % SQ).astype(jnp.float32)
        pair = ((coli % DH) // 2).astype(jnp.float32)
        inv = jnp.exp(pair * (-2.0 / DH) * jnp.log(10000.0))
        angle = pos * inv
        cos = jnp.cos(angle)
        sin = jnp.sin(angle)

        ri = lax.broadcasted_iota(jnp.int32, (HD_LOCAL, HD_LOCAL), 0)
        rj = lax.broadcasted_iota(jnp.int32, (HD_LOCAL, HD_LOCAL), 1)
        R = (jnp.where((rj == ri + 1) & (ri % 2 == 0), 1.0, 0.0)
             + jnp.where((rj == ri - 1) & (ri % 2 == 1), -1.0, 0.0))
        q = q * cos + jnp.dot(q, R, preferred_element_type=jnp.float32) * sin
        k = k * cos + jnp.dot(k, R, preferred_element_type=jnp.float32) * sin

        ctx_rows = []
        for b in range(B):
            ctx_heads = []
            for h in range(HQ_LOCAL):
                qs = q[b * SQ:(b + 1) * SQ, h * DH:(h + 1) * DH]
                ks = k[b * SQ:(b + 1) * SQ, h * DH:(h + 1) * DH]
                vs = v[b * SQ:(b + 1) * SQ, h * DH:(h + 1) * DH]
                sc = lax.dot_general(
                    qs, ks, (((1,), (1,)), ((), ())),
                    preferred_element_type=jnp.float32,
                ) * 0.125
                sc = sc - jnp.max(sc, axis=-1, keepdims=True)
                w = jnp.exp(sc)
                w = w / jnp.sum(w, axis=-1, keepdims=True)
                ctx_heads.append(lax.dot_general(
                    w, vs, (((1,), (0,)), ((), ())),
                    preferred_element_type=jnp.float32,
                ))
            ctx_rows.append(jnp.concatenate(ctx_heads, axis=1))
        ctx = jnp.concatenate(ctx_rows, axis=0)

        acc_ref[:, :] = jnp.dot(
            ctx.astype(jnp.bfloat16), wo_ref[:].astype(jnp.bfloat16),
            preferred_element_type=jnp.float32,
        )

        for s in range(N_STAGES):
            partner = jnp.bitwise_xor(my, 1 << s)
            send_ref[:, :] = acc_ref[:, :].astype(jnp.bfloat16)
            rdma = pltpu.make_async_remote_copy(
                src_ref=send_ref,
                dst_ref=recv_ref.at[s],
                send_sem=send_sems.at[s],
                recv_sem=recv_sems.at[s],
                device_id=(partner,),
                device_id_type=pl.DeviceIdType.MESH,
            )
            rdma.start()
            rdma.wait()
            acc_ref[:, :] = acc_ref[:, :] + recv_ref[s].astype(jnp.float32)

        out_ref[:, :, :] = acc_ref[:, :].reshape(B, SQ, D)

        @functools.partial(
            pl.run_scoped, second_barrier=pltpu.SemaphoreType.REGULAR
        )
        def _(second_barrier):
            for s in range(N_STAGES):
                partner = jnp.bitwise_xor(my, 1 << s)
                pl.semaphore_signal(
                    second_barrier, inc=1,
                    device_id=(partner,), device_id_type=pl.DeviceIdType.MESH,
                )
            pl.semaphore_wait(second_barrier, N_STAGES)

    return pl.pallas_call(
        body,
        out_shape=jax.ShapeDtypeStruct((B, SQ, D), jnp.float32),
        in_specs=[pl.BlockSpec(memory_space=pltpu.VMEM)] * 5,
        out_specs=pl.BlockSpec(memory_space=pltpu.VMEM),
        scratch_shapes=[
            pltpu.VMEM((ROWS, D), jnp.float32),
            pltpu.VMEM((ROWS, D), jnp.bfloat16),
            pltpu.VMEM((N_STAGES, ROWS, D), jnp.bfloat16),
            pltpu.SemaphoreType.DMA((N_STAGES,)),
            pltpu.SemaphoreType.DMA((N_STAGES,)),
        ],
        compiler_params=pltpu.CompilerParams(collective_id=0),
    )(x, Wq, Wk, Wv, Wo)


# device time: 31249 ns/iter; 2.1067x vs baseline; 1.4542x over previous
import functools

import jax
import jax.numpy as jnp
from jax import lax
from jax.experimental import pallas as pl
from jax.experimental.pallas import tpu as pltpu

N_DEV = 32

B, SQ, D = 2, 128, 512
HQ_LOCAL = 4
DH = 64
HD_LOCAL = HQ_LOCAL * DH
ROWS = B * SQ
RPS = ROWS // N_DEV


def kernel(x, Wq, Wk, Wv, Wo):
    def body(x_ref, wq_ref, wk_ref, wv_ref, wo_ref, out_ref,
             acc_ref, send_ref, rs_recv_ref, ag_send_ref, ag_recv_ref,
             rs_send_sems, rs_recv_sems, ag_send_sems, ag_recv_sems):
        my = lax.axis_index("i")

        barrier = pltpu.get_barrier_semaphore()
        for j in range(N_DEV):
            @pl.when(j != my)
            def _():
                pl.semaphore_signal(
                    barrier, inc=1,
                    device_id=(j,), device_id_type=pl.DeviceIdType.MESH,
                )
        pl.semaphore_wait(barrier, N_DEV - 1)

        xb = x_ref[:].reshape(ROWS, D).astype(jnp.bfloat16)
        q = jnp.dot(xb, wq_ref[:].astype(jnp.bfloat16),
                    preferred_element_type=jnp.float32)
        k = jnp.dot(xb, wk_ref[:].astype(jnp.bfloat16),
                    preferred_element_type=jnp.float32)
        v = jnp.dot(xb, wv_ref[:].astype(jnp.bfloat16),
                    preferred_element_type=jnp.float32)

        rowi = lax.broadcasted_iota(jnp.int32, (ROWS, HD_LOCAL), 0)
        coli = lax.broadcasted_iota(jnp.int32, (ROWS, HD_LOCAL), 1)
        pos = (rowi % SQ).astype(jnp.float32)
        pair = ((coli % DH) // 2).astype(jnp.float32)
        inv = jnp.exp(pair * (-2.0 / DH) * jnp.log(10000.0))
        angle = pos * inv
        cos = jnp.cos(angle)
        sin = jnp.sin(angle)

        ri = lax.broadcasted_iota(jnp.int32, (HD_LOCAL, HD_LOCAL), 0)
        rj = lax.broadcasted_iota(jnp.int32, (HD_LOCAL, HD_LOCAL), 1)
        R = (jnp.where((rj == ri + 1) & (ri % 2 == 0), 1.0, 0.0)
             + jnp.where((rj == ri - 1) & (ri % 2 == 1), -1.0, 0.0))
        q = q * cos + jnp.dot(q, R, preferred_element_type=jnp.float32) * sin
        k = k * cos + jnp.dot(k, R, preferred_element_type=jnp.float32) * sin

        ctx_rows = []
        for b in range(B):
            ctx_heads = []
            for h in range(HQ_LOCAL):
                qs = q[b * SQ:(b + 1) * SQ, h * DH:(h + 1) * DH]
                ks = k[b * SQ:(b + 1) * SQ, h * DH:(h + 1) * DH]
                vs = v[b * SQ:(b + 1) * SQ, h * DH:(h + 1) * DH]
                sc = lax.dot_general(
                    qs, ks, (((1,), (1,)), ((), ())),
                    preferred_element_type=jnp.float32,
                ) * 0.125
                sc = sc - jnp.max(sc, axis=-1, keepdims=True)
                w = jnp.exp(sc)
                w = w / jnp.sum(w, axis=-1, keepdims=True)
                ctx_heads.append(lax.dot_general(
                    w, vs, (((1,), (0,)), ((), ())),
                    preferred_element_type=jnp.float32,
                ))
            ctx_rows.append(jnp.concatenate(ctx_heads, axis=1))
        ctx = jnp.concatenate(ctx_rows, axis=0)

        acc_ref[:, :] = jnp.dot(
            ctx.astype(jnp.bfloat16), wo_ref[:].astype(jnp.bfloat16),
            preferred_element_type=jnp.float32,
        )

        send_ref[:, :] = acc_ref[:, :].astype(jnp.bfloat16)
        rs_recv_ref[my] = jnp.zeros((RPS, D), jnp.bfloat16)
        rs_rdmas = []
        for j in range(N_DEV):
            rdma = pltpu.make_async_remote_copy(
                src_ref=send_ref.at[pl.ds(j * RPS, RPS)],
                dst_ref=rs_recv_ref.at[my],
                send_sem=rs_send_sems.at[j],
                recv_sem=rs_recv_sems.at[my],
                device_id=(j,),
                device_id_type=pl.DeviceIdType.MESH,
            )
            rs_rdmas.append(rdma)

            @pl.when(j != my)
            def _():
                rdma.start()
        for j in range(N_DEV):
            recv = pltpu.make_async_remote_copy(
                src_ref=send_ref.at[pl.ds(0, RPS)],
                dst_ref=rs_recv_ref.at[j],
                send_sem=rs_send_sems.at[j],
                recv_sem=rs_recv_sems.at[j],
                device_id=(j,),
                device_id_type=pl.DeviceIdType.MESH,
            )

            @pl.when(j != my)
            def _():
                recv.wait_recv()

        red = (acc_ref[pl.ds(my * RPS, RPS), :]
               + jnp.sum(rs_recv_ref[:, :, :].astype(jnp.float32), axis=0))

        ag_send_ref[:, :] = red.astype(jnp.bfloat16)
        ag_recv_ref[my] = ag_send_ref[:, :]
        for j in range(N_DEV):
            rdma = pltpu.make_async_remote_copy(
                src_ref=ag_send_ref,
                dst_ref=ag_recv_ref.at[my],
                send_sem=ag_send_sems.at[j],
                recv_sem=ag_recv_sems.at[my],
                device_id=(j,),
                device_id_type=pl.DeviceIdType.MESH,
            )

            @pl.when(j != my)
            def _():
                rdma.start()
        for j in range(N_DEV):
            recv = pltpu.make_async_remote_copy(
                src_ref=ag_send_ref,
                dst_ref=ag_recv_ref.at[j],
                send_sem=ag_send_sems.at[j],
                recv_sem=ag_recv_sems.at[j],
                device_id=(j,),
                device_id_type=pl.DeviceIdType.MESH,
            )

            @pl.when(j != my)
            def _():
                recv.wait_recv()

        out_ref[:, :, :] = (
            ag_recv_ref[:, :, :].astype(jnp.float32).reshape(B, SQ, D)
        )

        for j in range(N_DEV):
            @pl.when(j != my)
            def _():
                rs_rdmas[j].wait_send()
        for j in range(N_DEV):
            drain = pltpu.make_async_remote_copy(
                src_ref=ag_send_ref,
                dst_ref=ag_recv_ref.at[j],
                send_sem=ag_send_sems.at[j],
                recv_sem=ag_recv_sems.at[j],
                device_id=(j,),
                device_id_type=pl.DeviceIdType.MESH,
            )

            @pl.when(j != my)
            def _():
                drain.wait_send()

        @functools.partial(
            pl.run_scoped, second_barrier=pltpu.SemaphoreType.REGULAR
        )
        def _(second_barrier):
            for j in range(N_DEV):
                @pl.when(j != my)
                def _():
                    pl.semaphore_signal(
                        second_barrier, inc=1,
                        device_id=(j,), device_id_type=pl.DeviceIdType.MESH,
                    )
            pl.semaphore_wait(second_barrier, N_DEV - 1)

    return pl.pallas_call(
        body,
        out_shape=jax.ShapeDtypeStruct((B, SQ, D), jnp.float32),
        in_specs=[pl.BlockSpec(memory_space=pltpu.VMEM)] * 5,
        out_specs=pl.BlockSpec(memory_space=pltpu.VMEM),
        scratch_shapes=[
            pltpu.VMEM((ROWS, D), jnp.float32),
            pltpu.VMEM((ROWS, D), jnp.bfloat16),
            pltpu.VMEM((N_DEV, RPS, D), jnp.bfloat16),
            pltpu.VMEM((RPS, D), jnp.bfloat16),
            pltpu.VMEM((N_DEV, RPS, D), jnp.bfloat16),
            pltpu.SemaphoreType.DMA((N_DEV,)),
            pltpu.SemaphoreType.DMA((N_DEV,)),
            pltpu.SemaphoreType.DMA((N_DEV,)),
            pltpu.SemaphoreType.DMA((N_DEV,)),
        ],
        compiler_params=pltpu.CompilerParams(collective_id=0),
    )(x, Wq, Wk, Wv, Wo)
